# baseline (speedup 1.0000x reference)
import functools
import jax
import jax.numpy as jnp
from jax import lax
from jax.experimental import pallas as pl
from jax.experimental.pallas import tpu as pltpu
from jax.experimental.pallas import tpu_sc as plsc

_BP = 512

_P = 50176
_K = 1024
_NW = 32
_PPW = _P // _NW
_R = 16
_NCHUNK = _PPW // _R


def _labels_body(xf_ref, c_ref, lab_ref, cb_ref, csqh_ref):
    @pl.when(pl.program_id(0) == 0)
    def _():
        c = c_ref[...]
        cb_ref[...] = c.astype(jnp.bfloat16)
        ones = jnp.ones((1, c.shape[1]), jnp.float32)
        csqh_ref[...] = 0.5 * jax.lax.dot_general(
            ones, c * c,
            dimension_numbers=(((1,), (1,)), ((), ())),
            preferred_element_type=jnp.float32,
            precision=jax.lax.Precision.HIGHEST,
        )

    d = jax.lax.dot_general(
        xf_ref[...].astype(jnp.bfloat16), cb_ref[...],
        dimension_numbers=(((1,), (1,)), ((), ())),
        preferred_element_type=jnp.float32,
    )
    s = d - csqh_ref[...]
    lab_ref[...] = jnp.argmax(s, axis=1).astype(jnp.int32)[:, None]


def _labels_tc(xf, cluster_centers):
    K, C = cluster_centers.shape
    grid = _P // _BP
    return pl.pallas_call(
        _labels_body,
        grid=(grid,),
        in_specs=[
            pl.BlockSpec((_BP, C), lambda i: (i, 0)),
            pl.BlockSpec((K, C), lambda i: (0, 0)),
        ],
        out_specs=pl.BlockSpec((_BP, 1), lambda i: (i, 0)),
        out_shape=jax.ShapeDtypeStruct((_P, 1), jnp.int32),
        scratch_shapes=[
            pltpu.VMEM((K, C), jnp.bfloat16),
            pltpu.VMEM((1, K), jnp.float32),
        ],
    )(xf, cluster_centers)


def _onehot_sc(labels):
    mesh = plsc.VectorSubcoreMesh(
        core_axis_name="c", subcore_axis_name="s", num_cores=2, num_subcores=16
    )

    @functools.partial(
        pl.kernel,
        mesh=mesh,
        compiler_params=pltpu.CompilerParams(
            use_tc_tiling_on_sc=False, needs_layout_passes=False
        ),
        out_type=jax.ShapeDtypeStruct((_P, _K), jnp.float32),
        scratch_types=[
            pltpu.VMEM((_PPW,), jnp.int32),
            pltpu.VMEM((_R, _K), jnp.float32),
            pltpu.VMEM((_R, _K), jnp.float32),
            pltpu.SemaphoreType.DMA,
            pltpu.SemaphoreType.DMA,
        ],
    )
    def k(labels_hbm, out_hbm, lab_v, buf_a, buf_b, sem_a, sem_b):
        wid = lax.axis_index("s") * 2 + lax.axis_index("c")
        base = wid * _PPW
        pltpu.sync_copy(labels_hbm.at[pl.ds(base, _PPW)], lab_v)

        zero16 = jnp.zeros((16,), jnp.float32)
        ones16 = jnp.ones((16,), jnp.float32)
        lane = lax.iota(jnp.int32, 16)
        bufs = (buf_a, buf_b)
        sems = (sem_a, sem_b)

        for buf in bufs:
            def zbody(i, _):
                r = i // (_K // 16)
                col = lax.rem(i, _K // 16) * 16
                buf[r, pl.ds(col, 16)] = zero16
                return 0
            lax.fori_loop(0, _R * _K // 16, zbody, 0)

        def scatter_chunk(buf, cidx, val16):
            labs = lab_v[pl.ds(cidx * _R, 16)]
            plsc.store_scatter(buf, [lane, labs], val16)

        def chunk_body(cidx, _):
            for par in range(2):
                @pl.when(lax.rem(cidx, 2) == par)
                def _():
                    buf, sem = bufs[par], sems[par]

                    @pl.when(cidx >= 2)
                    def _():
                        pltpu.make_async_copy(
                            buf, out_hbm.at[pl.ds(base + (cidx - 2) * _R, _R)],
                            sem,
                        ).wait()
                        scatter_chunk(buf, cidx - 2, zero16)

                    scatter_chunk(buf, cidx, ones16)
                    pltpu.async_copy(
                        buf, out_hbm.at[pl.ds(base + cidx * _R, _R)], sem,
                    )
            return 0

        lax.fori_loop(0, _NCHUNK, chunk_body, 0)

        for cidx in (_NCHUNK - 2, _NCHUNK - 1):
            buf, sem = bufs[cidx % 2], sems[cidx % 2]
            pltpu.make_async_copy(
                buf, out_hbm.at[pl.ds(base + cidx * _R, _R)], sem,
            ).wait()

    return k(labels)


def kernel(x, cluster_centers):
    H, W, C = x.shape
    K = cluster_centers.shape[0]
    xf = x.reshape(H * W, C)
    labels = _labels_tc(xf, cluster_centers).reshape(H * W)
    out = _onehot_sc(labels)
    return out.reshape(H, W, K)
